# baseline (speedup 1.0000x reference)
import jax
import jax.numpy as jnp
from jax import lax
from jax.experimental import pallas as pl
from jax.experimental.pallas import tpu as pltpu
from jax.experimental.pallas import tpu_sc as plsc

_VOCAB = 1000000
_EMBED_DIM = 32
_BATCH = 16384
_SEQ = 200

_NC = 2
_NS = 16
_NW = _NC * _NS
_ROWS = _BATCH * _SEQ
_PER_W = _ROWS // _NW
_CHUNK = 1600
_NCHUNKS = _PER_W // _CHUNK


def _sc_body(idx_hbm, table_hbm, out_hbm,
             idx_v0, idx_v1, rows_v0, rows_v1, gsem0, gsem1, wsem0, wsem1):
    wid = lax.axis_index("s") * _NC + lax.axis_index("c")
    base = wid * _PER_W
    lane = lax.iota(jnp.int32, 16)
    zero16 = jnp.zeros((16,), jnp.float32)
    idx_v = (idx_v0, idx_v1)
    rows_v = (rows_v0, rows_v1)
    gsem = (gsem0, gsem1)
    wsem = (wsem0, wsem1)

    def start_gather(g, b):
        start = base + g * _CHUNK
        pltpu.sync_copy(idx_hbm.at[pl.ds(start, _CHUNK)], idx_v[b])
        return pltpu.async_copy(table_hbm.at[idx_v[b]], rows_v[b], gsem[b])

    def writeback_slice(g, b):
        start = base + g * _CHUNK
        return (rows_v[b], out_hbm.at[pl.ds(start, _CHUNK)])

    def fix_and_writeback(g, b):
        def fix_body(i, _):
            idx16 = idx_v[b][pl.ds(i * 16, 16)]
            mask = idx16 == 0
            npad = plsc.all_reduce_population_count(mask)[0]

            @pl.when(npad > 0)
            def _():
                row16 = i * 16 + lane
                for c in range(_EMBED_DIM):
                    col16 = jnp.full((16,), c, jnp.int32)
                    plsc.store_scatter(rows_v[b], [row16, col16], zero16,
                                       mask=mask)
            return 0

        lax.fori_loop(0, _CHUNK // 16, fix_body, 0)
        src, dst = writeback_slice(g, b)
        return pltpu.async_copy(src, dst, wsem[b])

    def wait_wb(g, b):
        src, dst = writeback_slice(g, b)
        pltpu.make_async_copy(src, dst, wsem[b]).wait()

    def wait_gather(g, b):
        pltpu.make_async_copy(table_hbm.at[idx_v[b]], rows_v[b],
                              gsem[b]).wait()

    start_gather(0, 0)
    wait_gather(0, 0)
    start_gather(1, 1)
    fix_and_writeback(0, 0)

    def step(k, _):
        g = 2 * k + 1
        wait_gather(g, 1)
        wait_wb(g - 1, 0)
        start_gather(g + 1, 0)
        fix_and_writeback(g, 1)
        wait_gather(g + 1, 0)
        wait_wb(g, 1)
        @pl.when(g + 2 < _NCHUNKS)
        def _():
            start_gather(g + 2, 1)
        fix_and_writeback(g + 1, 0)
        return 0

    lax.fori_loop(0, (_NCHUNKS - 2) // 2, step, 0)

    g = _NCHUNKS - 1
    wait_gather(g, 1)
    fix_and_writeback(g, 1)
    wait_wb(g - 1, 0)
    wait_wb(g, 1)


def _eye_f32(n):
    r = lax.broadcasted_iota(jnp.int32, (n, n), 0)
    c = lax.broadcasted_iota(jnp.int32, (n, n), 1)
    return jnp.where(r == c, jnp.float32(1), jnp.float32(0))


def _mxu_transpose(y):
    ident = _eye_f32(y.shape[0])
    return lax.dot_general(y, ident, (((0,), (0,)), ((), ())),
                           precision=lax.Precision.HIGHEST)


def _tc_out_body(x_ref, o_ref):
    bb = o_ref.shape[1]
    x3 = x_ref[...].reshape(bb, _SEQ * _EMBED_DIM // 128, 128)
    for q in range(_SEQ * _EMBED_DIM // 128):
        o_ref[q * 128:(q + 1) * 128, :] = _mxu_transpose(x3[:, q, :])


def _tc_out_transpose(out128, bb):
    sd = _SEQ * _EMBED_DIM
    return pl.pallas_call(
        _tc_out_body,
        out_shape=jax.ShapeDtypeStruct((sd, _BATCH), jnp.float32),
        grid=(_BATCH // bb,),
        in_specs=[pl.BlockSpec((bb * sd // 128, 128), lambda i: (i, 0))],
        out_specs=pl.BlockSpec((sd, bb), lambda i: (0, i)),
    )(out128)


def _tc_table_body(x_ref, o_ref):
    y = _mxu_transpose(x_ref[...])
    y3 = y.reshape(-1, 4, _EMBED_DIM)
    o_ref[...] = jnp.concatenate(
        [y3[:, a, :] for a in range(4)], axis=1)


def _tc_table_to_rowmajor(tt, bn):
    return pl.pallas_call(
        _tc_table_body,
        out_shape=jax.ShapeDtypeStruct((_VOCAB // 4, 128), jnp.float32),
        grid=((_VOCAB + bn - 1) // bn,),
        in_specs=[pl.BlockSpec((_EMBED_DIM, bn), lambda i: (0, i))],
        out_specs=pl.BlockSpec((bn // 4, 128), lambda i: (i, 0)),
    )(tt)


@jax.jit
def kernel(x, table):
    idx = x.reshape(-1)
    table = _tc_table_to_rowmajor(jnp.transpose(table, (1, 0)),
                                  4096).reshape(_VOCAB, _EMBED_DIM)
    mesh = plsc.VectorSubcoreMesh(core_axis_name="c", subcore_axis_name="s")
    out = pl.kernel(
        _sc_body,
        out_type=jax.ShapeDtypeStruct((_ROWS, _EMBED_DIM), jnp.float32),
        mesh=mesh,
        compiler_params=pltpu.CompilerParams(needs_layout_passes=False,
                                             use_tc_tiling_on_sc=False),
        scratch_types=[
            pltpu.VMEM((_CHUNK,), jnp.int32),
            pltpu.VMEM((_CHUNK,), jnp.int32),
            pltpu.VMEM((_CHUNK, _EMBED_DIM), jnp.float32),
            pltpu.VMEM((_CHUNK, _EMBED_DIM), jnp.float32),
            pltpu.SemaphoreType.DMA,
            pltpu.SemaphoreType.DMA,
            pltpu.SemaphoreType.DMA,
            pltpu.SemaphoreType.DMA,
        ],
    )(idx, table)
    out2 = _tc_out_transpose(out.reshape(_ROWS // 4, 128), 128)
    return jnp.transpose(out2.reshape(_SEQ, _EMBED_DIM, _BATCH), (2, 0, 1))


# speedup vs baseline: 1.2568x; 1.2568x over previous
import jax
import jax.numpy as jnp
from jax import lax
from jax.experimental import pallas as pl
from jax.experimental.pallas import tpu as pltpu
from jax.experimental.pallas import tpu_sc as plsc

_VOCAB = 1000000
_EMBED_DIM = 32
_BATCH = 16384
_SEQ = 200

_NC = 2
_NS = 16
_NW = _NC * _NS
_ROWS = _BATCH * _SEQ
_PER_W = _ROWS // _NW
_CHUNK = 1600
_NCHUNKS = _PER_W // _CHUNK


def _sc_body(idx_hbm, table_hbm, out_hbm,
             idx_v0, idx_v1, rows_v0, rows_v1, gsem0, gsem1, wsem0, wsem1):
    wid = lax.axis_index("s") * _NC + lax.axis_index("c")
    base = wid * _PER_W
    lane = lax.iota(jnp.int32, 16)
    zero16 = jnp.zeros((16,), jnp.float32)
    idx_v = (idx_v0, idx_v1)
    rows_v = (rows_v0, rows_v1)
    gsem = (gsem0, gsem1)
    wsem = (wsem0, wsem1)

    def start_gather(g, b):
        start = base + g * _CHUNK
        pltpu.sync_copy(idx_hbm.at[pl.ds(start, _CHUNK)], idx_v[b])
        return pltpu.async_copy(table_hbm.at[idx_v[b]], rows_v[b], gsem[b])

    def writeback_slice(g, b):
        start = base + g * _CHUNK
        return (rows_v[b], out_hbm.at[pl.ds(start, _CHUNK)])

    def fix_and_writeback(g, b):
        def fix_body(i, _):
            idx16 = idx_v[b][pl.ds(i * 16, 16)]
            mask = idx16 == 0
            npad = plsc.all_reduce_population_count(mask)[0]

            @pl.when(npad > 0)
            def _():
                row16 = i * 16 + lane
                for c in range(_EMBED_DIM):
                    col16 = jnp.full((16,), c, jnp.int32)
                    plsc.store_scatter(rows_v[b], [row16, col16], zero16,
                                       mask=mask)
            return 0

        lax.fori_loop(0, _CHUNK // 16, fix_body, 0)
        src, dst = writeback_slice(g, b)
        return pltpu.async_copy(src, dst, wsem[b])

    def wait_wb(g, b):
        src, dst = writeback_slice(g, b)
        pltpu.make_async_copy(src, dst, wsem[b]).wait()

    def wait_gather(g, b):
        pltpu.make_async_copy(table_hbm.at[idx_v[b]], rows_v[b],
                              gsem[b]).wait()

    start_gather(0, 0)
    wait_gather(0, 0)
    start_gather(1, 1)
    fix_and_writeback(0, 0)

    def step(k, _):
        g = 2 * k + 1
        wait_gather(g, 1)
        wait_wb(g - 1, 0)
        start_gather(g + 1, 0)
        fix_and_writeback(g, 1)
        wait_gather(g + 1, 0)
        wait_wb(g, 1)
        @pl.when(g + 2 < _NCHUNKS)
        def _():
            start_gather(g + 2, 1)
        fix_and_writeback(g + 1, 0)
        return 0

    lax.fori_loop(0, (_NCHUNKS - 2) // 2, step, 0)

    g = _NCHUNKS - 1
    wait_gather(g, 1)
    fix_and_writeback(g, 1)
    wait_wb(g - 1, 0)
    wait_wb(g, 1)


def _eye_f32(n):
    r = lax.broadcasted_iota(jnp.int32, (n, n), 0)
    c = lax.broadcasted_iota(jnp.int32, (n, n), 1)
    return jnp.where(r == c, jnp.float32(1), jnp.float32(0))


def _mxu_transpose(y):
    ident = _eye_f32(y.shape[0])
    return lax.dot_general(y, ident, (((0,), (0,)), ((), ())),
                           precision=lax.Precision.HIGHEST)


def _tc_out_body(x_ref, o_ref):
    bb = o_ref.shape[1]
    x3 = x_ref[...].reshape(bb, _SEQ * _EMBED_DIM // 128, 128)
    for q in range(_SEQ * _EMBED_DIM // 128):
        o_ref[q * 128:(q + 1) * 128, :] = _mxu_transpose(x3[:, q, :])


def _tc_out_transpose(out128, bb):
    sd = _SEQ * _EMBED_DIM
    return pl.pallas_call(
        _tc_out_body,
        out_shape=jax.ShapeDtypeStruct((sd, _BATCH), jnp.float32),
        grid=(_BATCH // bb,),
        in_specs=[pl.BlockSpec((bb * sd // 128, 128), lambda i: (i, 0))],
        out_specs=pl.BlockSpec((sd, bb), lambda i: (0, i)),
    )(out128)


def _tc_table_body(x_ref, o_ref):
    y = jnp.transpose(x_ref[...], (1, 0))
    y3 = y.reshape(-1, 4, _EMBED_DIM)
    o_ref[...] = jnp.concatenate(
        [y3[:, a, :] for a in range(4)], axis=1)


def _tc_table_to_rowmajor(tt, bn):
    return pl.pallas_call(
        _tc_table_body,
        out_shape=jax.ShapeDtypeStruct((_VOCAB // 4, 128), jnp.float32),
        grid=((_VOCAB + bn - 1) // bn,),
        in_specs=[pl.BlockSpec((_EMBED_DIM, bn), lambda i: (0, i))],
        out_specs=pl.BlockSpec((bn // 4, 128), lambda i: (i, 0)),
    )(tt)


@jax.jit
def kernel(x, table):
    idx = x.reshape(-1)
    table = _tc_table_to_rowmajor(jnp.transpose(table, (1, 0)),
                                  4096).reshape(_VOCAB, _EMBED_DIM)
    mesh = plsc.VectorSubcoreMesh(core_axis_name="c", subcore_axis_name="s")
    out = pl.kernel(
        _sc_body,
        out_type=jax.ShapeDtypeStruct((_ROWS, _EMBED_DIM), jnp.float32),
        mesh=mesh,
        compiler_params=pltpu.CompilerParams(needs_layout_passes=False,
                                             use_tc_tiling_on_sc=False),
        scratch_types=[
            pltpu.VMEM((_CHUNK,), jnp.int32),
            pltpu.VMEM((_CHUNK,), jnp.int32),
            pltpu.VMEM((_CHUNK, _EMBED_DIM), jnp.float32),
            pltpu.VMEM((_CHUNK, _EMBED_DIM), jnp.float32),
            pltpu.SemaphoreType.DMA,
            pltpu.SemaphoreType.DMA,
            pltpu.SemaphoreType.DMA,
            pltpu.SemaphoreType.DMA,
        ],
    )(idx, table)
    out2 = _tc_out_transpose(out.reshape(_ROWS // 4, 128), 128)
    return jnp.transpose(out2.reshape(_SEQ, _EMBED_DIM, _BATCH), (2, 0, 1))


# speedup vs baseline: 1.3485x; 1.0729x over previous
import jax
import jax.numpy as jnp
from jax import lax
from jax.experimental import pallas as pl
from jax.experimental.pallas import tpu as pltpu
from jax.experimental.pallas import tpu_sc as plsc

_VOCAB = 1000000
_EMBED_DIM = 32
_BATCH = 16384
_SEQ = 200

_NC = 2
_NS = 16
_NW = _NC * _NS
_ROWS = _BATCH * _SEQ
_PER_W = _ROWS // _NW
_CHUNK = 1600
_NCHUNKS = _PER_W // _CHUNK


def _make_sc_body(idx_off, per_w):
  nchunks = per_w // _CHUNK

  def _sc_body(idx_hbm, table_hbm, out_hbm,
               idx_v0, idx_v1, rows_v0, rows_v1, gsem0, gsem1, wsem0, wsem1):
    wid = lax.axis_index("s") * _NC + lax.axis_index("c")
    base = wid * per_w
    ibase = idx_off + base
    lane = lax.iota(jnp.int32, 16)
    zero16 = jnp.zeros((16,), jnp.float32)
    idx_v = (idx_v0, idx_v1)
    rows_v = (rows_v0, rows_v1)
    gsem = (gsem0, gsem1)
    wsem = (wsem0, wsem1)

    def start_gather(g, b):
        start = ibase + g * _CHUNK
        pltpu.sync_copy(idx_hbm.at[pl.ds(start, _CHUNK)], idx_v[b])
        return pltpu.async_copy(table_hbm.at[idx_v[b]], rows_v[b], gsem[b])

    def writeback_slice(g, b):
        start = base + g * _CHUNK
        return (rows_v[b], out_hbm.at[pl.ds(start, _CHUNK)])

    def fix_and_writeback(g, b):
        def fix_body(i, _):
            idx16 = idx_v[b][pl.ds(i * 16, 16)]
            mask = idx16 == 0
            npad = plsc.all_reduce_population_count(mask)[0]

            @pl.when(npad > 0)
            def _():
                row16 = i * 16 + lane
                for c in range(_EMBED_DIM):
                    col16 = jnp.full((16,), c, jnp.int32)
                    plsc.store_scatter(rows_v[b], [row16, col16], zero16,
                                       mask=mask)
            return 0

        lax.fori_loop(0, _CHUNK // 16, fix_body, 0)
        src, dst = writeback_slice(g, b)
        return pltpu.async_copy(src, dst, wsem[b])

    def wait_wb(g, b):
        src, dst = writeback_slice(g, b)
        pltpu.make_async_copy(src, dst, wsem[b]).wait()

    def wait_gather(g, b):
        pltpu.make_async_copy(table_hbm.at[idx_v[b]], rows_v[b],
                              gsem[b]).wait()

    start_gather(0, 0)
    wait_gather(0, 0)
    start_gather(1, 1)
    fix_and_writeback(0, 0)

    def step(k, _):
        g = 2 * k + 1
        wait_gather(g, 1)
        wait_wb(g - 1, 0)
        start_gather(g + 1, 0)
        fix_and_writeback(g, 1)
        wait_gather(g + 1, 0)
        wait_wb(g, 1)
        @pl.when(g + 2 < nchunks)
        def _():
            start_gather(g + 2, 1)
        fix_and_writeback(g + 1, 0)
        return 0

    lax.fori_loop(0, (nchunks - 2) // 2, step, 0)

    g = nchunks - 1
    wait_gather(g, 1)
    fix_and_writeback(g, 1)
    wait_wb(g - 1, 0)
    wait_wb(g, 1)

  return _sc_body


def _eye_f32(n):
    r = lax.broadcasted_iota(jnp.int32, (n, n), 0)
    c = lax.broadcasted_iota(jnp.int32, (n, n), 1)
    return jnp.where(r == c, jnp.float32(1), jnp.float32(0))


def _mxu_transpose(y):
    ident = _eye_f32(y.shape[0])
    return lax.dot_general(y, ident, (((0,), (0,)), ((), ())),
                           precision=lax.Precision.HIGHEST)


def _tc_out_body(x_ref, o_ref):
    bb = o_ref.shape[1]
    x3 = x_ref[...].reshape(bb, _SEQ * _EMBED_DIM // 128, 128)
    for q in range(_SEQ * _EMBED_DIM // 128):
        o_ref[q * 128:(q + 1) * 128, :] = _mxu_transpose(x3[:, q, :])


def _tc_out_body2(x_ref, prev_ref, o_ref):
    del prev_ref
    _tc_out_body(x_ref, o_ref)


def _tc_out_transpose(outa128, outb128, bb):
    sd = _SEQ * _EMBED_DIM
    hblk = _BATCH // 2 // bb
    t1 = pl.pallas_call(
        _tc_out_body,
        out_shape=jax.ShapeDtypeStruct((sd, _BATCH), jnp.float32),
        grid=(hblk,),
        in_specs=[pl.BlockSpec((bb * sd // 128, 128), lambda i: (i, 0))],
        out_specs=pl.BlockSpec((sd, bb), lambda i: (0, i)),
    )(outa128)
    return pl.pallas_call(
        _tc_out_body2,
        out_shape=jax.ShapeDtypeStruct((sd, _BATCH), jnp.float32),
        grid=(hblk,),
        in_specs=[pl.BlockSpec((bb * sd // 128, 128), lambda i: (i, 0)),
                  pl.BlockSpec((8, 128), lambda i: (0, 0))],
        out_specs=pl.BlockSpec((sd, bb), lambda i: (0, i + hblk)),
        input_output_aliases={1: 0},
    )(outb128, t1)


def _tc_table_body(x_ref, o_ref):
    y = jnp.transpose(x_ref[...], (1, 0))
    y3 = y.reshape(-1, 4, _EMBED_DIM)
    o_ref[...] = jnp.concatenate(
        [y3[:, a, :] for a in range(4)], axis=1)


def _tc_table_to_rowmajor(tt, bn):
    return pl.pallas_call(
        _tc_table_body,
        out_shape=jax.ShapeDtypeStruct((_VOCAB // 4, 128), jnp.float32),
        grid=((_VOCAB + bn - 1) // bn,),
        in_specs=[pl.BlockSpec((_EMBED_DIM, bn), lambda i: (0, i))],
        out_specs=pl.BlockSpec((bn // 4, 128), lambda i: (i, 0)),
    )(tt)


@jax.jit
def kernel(x, table):
    idx = x.reshape(-1)
    table = _tc_table_to_rowmajor(jnp.transpose(table, (1, 0)),
                                  4096).reshape(_VOCAB, _EMBED_DIM)
    mesh = plsc.VectorSubcoreMesh(core_axis_name="c", subcore_axis_name="s")

    def gather_half(idx_off):
        return pl.kernel(
            _make_sc_body(idx_off, _ROWS // 2 // _NW),
            out_type=jax.ShapeDtypeStruct((_ROWS // 2, _EMBED_DIM),
                                          jnp.float32),
            mesh=mesh,
            compiler_params=pltpu.CompilerParams(needs_layout_passes=False,
                                                 use_tc_tiling_on_sc=False),
            scratch_types=[
                pltpu.VMEM((_CHUNK,), jnp.int32),
                pltpu.VMEM((_CHUNK,), jnp.int32),
                pltpu.VMEM((_CHUNK, _EMBED_DIM), jnp.float32),
                pltpu.VMEM((_CHUNK, _EMBED_DIM), jnp.float32),
                pltpu.SemaphoreType.DMA,
                pltpu.SemaphoreType.DMA,
                pltpu.SemaphoreType.DMA,
                pltpu.SemaphoreType.DMA,
            ],
        )(idx, table)

    outa = gather_half(0)
    outb = gather_half(_ROWS // 2)
    out2 = _tc_out_transpose(outa.reshape(_ROWS // 8, 128),
                             outb.reshape(_ROWS // 8, 128), 128)
    return jnp.transpose(out2.reshape(_SEQ, _EMBED_DIM, _BATCH), (2, 0, 1))


# speedup vs baseline: 1.3619x; 1.0100x over previous
import jax
import jax.numpy as jnp
from jax import lax
from jax.experimental import pallas as pl
from jax.experimental.pallas import tpu as pltpu
from jax.experimental.pallas import tpu_sc as plsc

_VOCAB = 1000000
_EMBED_DIM = 32
_BATCH = 16384
_SEQ = 200

_NC = 2
_NS = 16
_NW = _NC * _NS
_ROWS = _BATCH * _SEQ
_PER_W = _ROWS // _NW
_CHUNK = 1600
_NCHUNKS = _PER_W // _CHUNK


def _make_sc_body(idx_off, per_w):
  nchunks = per_w // _CHUNK

  def _sc_body(idx_hbm, table_hbm, out_hbm,
               idx_v0, idx_v1, rows_v0, rows_v1, gsem0, gsem1, wsem0, wsem1):
    wid = lax.axis_index("s") * _NC + lax.axis_index("c")
    base = wid * per_w
    ibase = idx_off + base
    lane = lax.iota(jnp.int32, 16)
    zero16 = jnp.zeros((16,), jnp.float32)
    idx_v = (idx_v0, idx_v1)
    rows_v = (rows_v0, rows_v1)
    gsem = (gsem0, gsem1)
    wsem = (wsem0, wsem1)

    def start_gather(g, b):
        start = ibase + g * _CHUNK
        pltpu.sync_copy(idx_hbm.at[pl.ds(start, _CHUNK)], idx_v[b])
        return pltpu.async_copy(table_hbm.at[idx_v[b]], rows_v[b], gsem[b])

    def writeback_slice(g, b):
        start = base + g * _CHUNK
        return (rows_v[b], out_hbm.at[pl.ds(start, _CHUNK)])

    def fix_and_writeback(g, b):
        def fix_body(i, _):
            idx16 = idx_v[b][pl.ds(i * 16, 16)]
            mask = idx16 == 0
            npad = plsc.all_reduce_population_count(mask)[0]

            @pl.when(npad > 0)
            def _():
                row16 = i * 16 + lane
                for c in range(_EMBED_DIM):
                    col16 = jnp.full((16,), c, jnp.int32)
                    plsc.store_scatter(rows_v[b], [row16, col16], zero16,
                                       mask=mask)
            return 0

        lax.fori_loop(0, _CHUNK // 16, fix_body, 0)
        src, dst = writeback_slice(g, b)
        return pltpu.async_copy(src, dst, wsem[b])

    def wait_wb(g, b):
        src, dst = writeback_slice(g, b)
        pltpu.make_async_copy(src, dst, wsem[b]).wait()

    def wait_gather(g, b):
        pltpu.make_async_copy(table_hbm.at[idx_v[b]], rows_v[b],
                              gsem[b]).wait()

    start_gather(0, 0)
    wait_gather(0, 0)
    start_gather(1, 1)
    fix_and_writeback(0, 0)

    def step(k, _):
        g = 2 * k + 1
        wait_gather(g, 1)
        wait_wb(g - 1, 0)
        start_gather(g + 1, 0)
        fix_and_writeback(g, 1)
        wait_gather(g + 1, 0)
        wait_wb(g, 1)
        @pl.when(g + 2 < nchunks)
        def _():
            start_gather(g + 2, 1)
        fix_and_writeback(g + 1, 0)
        return 0

    lax.fori_loop(0, (nchunks - 2) // 2, step, 0)

    g = nchunks - 1
    wait_gather(g, 1)
    fix_and_writeback(g, 1)
    wait_wb(g - 1, 0)
    wait_wb(g, 1)

  return _sc_body


def _eye_f32(n):
    r = lax.broadcasted_iota(jnp.int32, (n, n), 0)
    c = lax.broadcasted_iota(jnp.int32, (n, n), 1)
    return jnp.where(r == c, jnp.float32(1), jnp.float32(0))


def _mxu_transpose(y):
    ident = _eye_f32(y.shape[0])
    return lax.dot_general(y, ident, (((0,), (0,)), ((), ())),
                           precision=lax.Precision.HIGHEST)


def _tc_out_body(x_ref, o_ref):
    bb = o_ref.shape[1]
    x3 = x_ref[...].reshape(bb, _SEQ * _EMBED_DIM // 128, 128)
    for q in range(_SEQ * _EMBED_DIM // 128):
        o_ref[q * 128:(q + 1) * 128, :] = _mxu_transpose(x3[:, q, :])


def _tc_out_body2(x_ref, prev_ref, o_ref):
    del prev_ref
    _tc_out_body(x_ref, o_ref)


def _tc_out_transpose(outa128, outb128, bb):
    sd = _SEQ * _EMBED_DIM
    hblk = _BATCH // 2 // bb
    t1 = pl.pallas_call(
        _tc_out_body,
        out_shape=jax.ShapeDtypeStruct((sd, _BATCH), jnp.float32),
        grid=(hblk,),
        in_specs=[pl.BlockSpec((bb * sd // 128, 128), lambda i: (i, 0))],
        out_specs=pl.BlockSpec((sd, bb), lambda i: (0, i)),
    )(outa128)
    return pl.pallas_call(
        _tc_out_body2,
        out_shape=jax.ShapeDtypeStruct((sd, _BATCH), jnp.float32),
        grid=(hblk,),
        in_specs=[pl.BlockSpec((bb * sd // 128, 128), lambda i: (i, 0)),
                  pl.BlockSpec((8, 128), lambda i: (0, 0))],
        out_specs=pl.BlockSpec((sd, bb), lambda i: (0, i + hblk)),
        input_output_aliases={1: 0},
    )(outb128, t1)


def _tc_table_body(x_ref, o_ref):
    y = jnp.transpose(x_ref[...], (1, 0))
    y3 = y.reshape(-1, 4, _EMBED_DIM)
    o_ref[...] = jnp.concatenate(
        [y3[:, a, :] for a in range(4)], axis=1)


def _tc_table_to_rowmajor(tt, bn):
    return pl.pallas_call(
        _tc_table_body,
        out_shape=jax.ShapeDtypeStruct((_VOCAB // 4, 128), jnp.float32),
        grid=((_VOCAB + bn - 1) // bn,),
        in_specs=[pl.BlockSpec((_EMBED_DIM, bn), lambda i: (0, i))],
        out_specs=pl.BlockSpec((bn // 4, 128), lambda i: (i, 0)),
    )(tt)


@jax.jit
def kernel(x, table):
    idx = x.reshape(-1)
    table = _tc_table_to_rowmajor(jnp.transpose(table, (1, 0)),
                                  8192).reshape(_VOCAB, _EMBED_DIM)
    mesh = plsc.VectorSubcoreMesh(core_axis_name="c", subcore_axis_name="s")

    def gather_half(idx_off):
        return pl.kernel(
            _make_sc_body(idx_off, _ROWS // 2 // _NW),
            out_type=jax.ShapeDtypeStruct((_ROWS // 2, _EMBED_DIM),
                                          jnp.float32),
            mesh=mesh,
            compiler_params=pltpu.CompilerParams(needs_layout_passes=False,
                                                 use_tc_tiling_on_sc=False),
            scratch_types=[
                pltpu.VMEM((_CHUNK,), jnp.int32),
                pltpu.VMEM((_CHUNK,), jnp.int32),
                pltpu.VMEM((_CHUNK, _EMBED_DIM), jnp.float32),
                pltpu.VMEM((_CHUNK, _EMBED_DIM), jnp.float32),
                pltpu.SemaphoreType.DMA,
                pltpu.SemaphoreType.DMA,
                pltpu.SemaphoreType.DMA,
                pltpu.SemaphoreType.DMA,
            ],
        )(idx, table)

    outa = gather_half(0)
    outb = gather_half(_ROWS // 2)
    out2 = _tc_out_transpose(outa.reshape(_ROWS // 8, 128),
                             outb.reshape(_ROWS // 8, 128), 128)
    return jnp.transpose(out2.reshape(_SEQ, _EMBED_DIM, _BATCH), (2, 0, 1))


# speedup vs baseline: 1.4096x; 1.0350x over previous
import jax
import jax.numpy as jnp
from jax import lax
from jax.experimental import pallas as pl
from jax.experimental.pallas import tpu as pltpu
from jax.experimental.pallas import tpu_sc as plsc

_VOCAB = 1000000
_EMBED_DIM = 32
_BATCH = 16384
_SEQ = 200

_NC = 2
_NS = 16
_NW = _NC * _NS
_ROWS = _BATCH * _SEQ
_PER_W = _ROWS // _NW
_CHUNK = 1600
_NCHUNKS = _PER_W // _CHUNK


def _make_sc_body(idx_off, per_w):
  nchunks = per_w // _CHUNK

  def _sc_body(idx_hbm, table_hbm, out_hbm,
               idx_v0, idx_v1, rows_v0, rows_v1, gsem0, gsem1, wsem0, wsem1):
    wid = lax.axis_index("s") * _NC + lax.axis_index("c")
    base = wid * per_w
    ibase = idx_off + base
    lane = lax.iota(jnp.int32, 16)
    zero16 = jnp.zeros((16,), jnp.float32)
    idx_v = (idx_v0, idx_v1)
    rows_v = (rows_v0, rows_v1)
    gsem = (gsem0, gsem1)
    wsem = (wsem0, wsem1)

    def start_gather(g, b):
        start = ibase + g * _CHUNK
        pltpu.sync_copy(idx_hbm.at[pl.ds(start, _CHUNK)], idx_v[b])
        return pltpu.async_copy(table_hbm.at[idx_v[b]], rows_v[b], gsem[b])

    def writeback_slice(g, b):
        start = base + g * _CHUNK
        return (rows_v[b], out_hbm.at[pl.ds(start, _CHUNK)])

    def fix_and_writeback(g, b):
        def fix_body(i, _):
            idx16 = idx_v[b][pl.ds(i * 16, 16)]
            mask = idx16 == 0
            npad = plsc.all_reduce_population_count(mask)[0]

            @pl.when(npad > 0)
            def _():
                row16 = i * 16 + lane
                for c in range(_EMBED_DIM):
                    col16 = jnp.full((16,), c, jnp.int32)
                    plsc.store_scatter(rows_v[b], [row16, col16], zero16,
                                       mask=mask)
            return 0

        lax.fori_loop(0, _CHUNK // 16, fix_body, 0)
        src, dst = writeback_slice(g, b)
        return pltpu.async_copy(src, dst, wsem[b])

    def wait_wb(g, b):
        src, dst = writeback_slice(g, b)
        pltpu.make_async_copy(src, dst, wsem[b]).wait()

    def wait_gather(g, b):
        pltpu.make_async_copy(table_hbm.at[idx_v[b]], rows_v[b],
                              gsem[b]).wait()

    start_gather(0, 0)
    wait_gather(0, 0)
    start_gather(1, 1)
    fix_and_writeback(0, 0)

    def step(k, _):
        g = 2 * k + 1
        wait_gather(g, 1)
        wait_wb(g - 1, 0)
        start_gather(g + 1, 0)
        fix_and_writeback(g, 1)
        wait_gather(g + 1, 0)
        wait_wb(g, 1)
        @pl.when(g + 2 < nchunks)
        def _():
            start_gather(g + 2, 1)
        fix_and_writeback(g + 1, 0)
        return 0

    lax.fori_loop(0, (nchunks - 2) // 2, step, 0)

    g = nchunks - 1
    wait_gather(g, 1)
    fix_and_writeback(g, 1)
    wait_wb(g - 1, 0)
    wait_wb(g, 1)

  return _sc_body


def _eye_f32(n):
    r = lax.broadcasted_iota(jnp.int32, (n, n), 0)
    c = lax.broadcasted_iota(jnp.int32, (n, n), 1)
    return jnp.where(r == c, jnp.float32(1), jnp.float32(0))


def _mxu_transpose(y):
    ident = _eye_f32(y.shape[0])
    return lax.dot_general(y, ident, (((0,), (0,)), ((), ())),
                           precision=lax.Precision.HIGHEST)


def _tc_out_body(x_ref, o_ref):
    bb = o_ref.shape[1]
    x3 = x_ref[...].reshape(bb, _SEQ * _EMBED_DIM // 128, 128)
    for q in range(_SEQ * _EMBED_DIM // 128):
        o_ref[q * 128:(q + 1) * 128, :] = _mxu_transpose(x3[:, q, :])


def _tc_out_body2(x_ref, prev_ref, o_ref):
    del prev_ref
    _tc_out_body(x_ref, o_ref)


def _tc_out_transpose(pieces, bb):
    sd = _SEQ * _EMBED_DIM
    np_ = len(pieces)
    hblk = _BATCH // np_ // bb
    t = pl.pallas_call(
        _tc_out_body,
        out_shape=jax.ShapeDtypeStruct((sd, _BATCH), jnp.float32),
        grid=(hblk,),
        in_specs=[pl.BlockSpec((bb * sd // 128, 128), lambda i: (i, 0))],
        out_specs=pl.BlockSpec((sd, bb), lambda i: (0, i)),
    )(pieces[0])
    for k in range(1, np_):
        t = pl.pallas_call(
            _tc_out_body2,
            out_shape=jax.ShapeDtypeStruct((sd, _BATCH), jnp.float32),
            grid=(hblk,),
            in_specs=[pl.BlockSpec((bb * sd // 128, 128), lambda i: (i, 0)),
                      pl.BlockSpec((8, 128), lambda i: (0, 0))],
            out_specs=pl.BlockSpec((sd, bb),
                                   lambda i, k=k: (0, i + k * hblk)),
            input_output_aliases={1: 0},
        )(pieces[k], t)
    return t


def _tc_table_body(x_ref, o_ref):
    y = jnp.transpose(x_ref[...], (1, 0))
    y3 = y.reshape(-1, 4, _EMBED_DIM)
    o_ref[...] = jnp.concatenate(
        [y3[:, a, :] for a in range(4)], axis=1)


def _tc_table_to_rowmajor(tt, bn):
    return pl.pallas_call(
        _tc_table_body,
        out_shape=jax.ShapeDtypeStruct((_VOCAB // 4, 128), jnp.float32),
        grid=((_VOCAB + bn - 1) // bn,),
        in_specs=[pl.BlockSpec((_EMBED_DIM, bn), lambda i: (0, i))],
        out_specs=pl.BlockSpec((bn // 4, 128), lambda i: (i, 0)),
    )(tt)


@jax.jit
def kernel(x, table):
    idx = x.reshape(-1)
    table = _tc_table_to_rowmajor(jnp.transpose(table, (1, 0)),
                                  8192).reshape(_VOCAB, _EMBED_DIM)
    mesh = plsc.VectorSubcoreMesh(core_axis_name="c", subcore_axis_name="s")

    def gather_half(idx_off):
        return pl.kernel(
            _make_sc_body(idx_off, _ROWS // 4 // _NW),
            out_type=jax.ShapeDtypeStruct((_ROWS // 4, _EMBED_DIM),
                                          jnp.float32),
            mesh=mesh,
            compiler_params=pltpu.CompilerParams(needs_layout_passes=False,
                                                 use_tc_tiling_on_sc=False),
            scratch_types=[
                pltpu.VMEM((_CHUNK,), jnp.int32),
                pltpu.VMEM((_CHUNK,), jnp.int32),
                pltpu.VMEM((_CHUNK, _EMBED_DIM), jnp.float32),
                pltpu.VMEM((_CHUNK, _EMBED_DIM), jnp.float32),
                pltpu.SemaphoreType.DMA,
                pltpu.SemaphoreType.DMA,
                pltpu.SemaphoreType.DMA,
                pltpu.SemaphoreType.DMA,
            ],
        )(idx, table)

    np_ = 4
    pieces = [gather_half(k * _ROWS // np_).reshape(_ROWS // np_ // 4, 128)
              for k in range(np_)]
    out2 = _tc_out_transpose(pieces, 128)
    return jnp.transpose(out2.reshape(_SEQ, _EMBED_DIM, _BATCH), (2, 0, 1))


# speedup vs baseline: 1.4284x; 1.0133x over previous
import jax
import jax.numpy as jnp
from jax import lax
from jax.experimental import pallas as pl
from jax.experimental.pallas import tpu as pltpu
from jax.experimental.pallas import tpu_sc as plsc

_VOCAB = 1000000
_EMBED_DIM = 32
_BATCH = 16384
_SEQ = 200

_NC = 2
_NS = 16
_NW = _NC * _NS
_ROWS = _BATCH * _SEQ
_PER_W = _ROWS // _NW
_CHUNK = 1600
_NCHUNKS = _PER_W // _CHUNK


def _make_sc_body(idx_off, per_w):
  nchunks = per_w // _CHUNK

  def _sc_body(idx_hbm, table_hbm, out_hbm,
               idx_v0, idx_v1, rows_v0, rows_v1, gsem0, gsem1, wsem0, wsem1):
    wid = lax.axis_index("s") * _NC + lax.axis_index("c")
    base = wid * per_w
    ibase = idx_off + base
    lane = lax.iota(jnp.int32, 16)
    zero16 = jnp.zeros((16,), jnp.float32)
    idx_v = (idx_v0, idx_v1)
    rows_v = (rows_v0, rows_v1)
    gsem = (gsem0, gsem1)
    wsem = (wsem0, wsem1)

    def start_gather(g, b):
        start = ibase + g * _CHUNK
        pltpu.sync_copy(idx_hbm.at[pl.ds(start, _CHUNK)], idx_v[b])
        return pltpu.async_copy(table_hbm.at[idx_v[b]], rows_v[b], gsem[b])

    def writeback_slice(g, b):
        start = base + g * _CHUNK
        return (rows_v[b], out_hbm.at[pl.ds(start, _CHUNK)])

    def fix_and_writeback(g, b):
        def fix_body(i, _):
            idx16 = idx_v[b][pl.ds(i * 16, 16)]
            mask = idx16 == 0
            npad = plsc.all_reduce_population_count(mask)[0]

            @pl.when(npad > 0)
            def _():
                row16 = i * 16 + lane
                for c in range(_EMBED_DIM):
                    col16 = jnp.full((16,), c, jnp.int32)
                    plsc.store_scatter(rows_v[b], [row16, col16], zero16,
                                       mask=mask)
            return 0

        lax.fori_loop(0, _CHUNK // 16, fix_body, 0)
        src, dst = writeback_slice(g, b)
        return pltpu.async_copy(src, dst, wsem[b])

    def wait_wb(g, b):
        src, dst = writeback_slice(g, b)
        pltpu.make_async_copy(src, dst, wsem[b]).wait()

    def wait_gather(g, b):
        pltpu.make_async_copy(table_hbm.at[idx_v[b]], rows_v[b],
                              gsem[b]).wait()

    start_gather(0, 0)
    wait_gather(0, 0)
    start_gather(1, 1)
    fix_and_writeback(0, 0)

    def step(k, _):
        g = 2 * k + 1
        wait_gather(g, 1)
        wait_wb(g - 1, 0)
        start_gather(g + 1, 0)
        fix_and_writeback(g, 1)
        wait_gather(g + 1, 0)
        wait_wb(g, 1)
        @pl.when(g + 2 < nchunks)
        def _():
            start_gather(g + 2, 1)
        fix_and_writeback(g + 1, 0)
        return 0

    lax.fori_loop(0, (nchunks - 2) // 2, step, 0)

    g = nchunks - 1
    wait_gather(g, 1)
    fix_and_writeback(g, 1)
    wait_wb(g - 1, 0)
    wait_wb(g, 1)

  return _sc_body


def _eye_f32(n):
    r = lax.broadcasted_iota(jnp.int32, (n, n), 0)
    c = lax.broadcasted_iota(jnp.int32, (n, n), 1)
    return jnp.where(r == c, jnp.float32(1), jnp.float32(0))


def _mxu_transpose(y):
    ident = _eye_f32(y.shape[0])
    return lax.dot_general(y, ident, (((0,), (0,)), ((), ())),
                           precision=lax.Precision.HIGHEST)


def _tc_out_body(x_ref, o_ref):
    bb = o_ref.shape[1]
    x3 = x_ref[...].reshape(bb, _SEQ * _EMBED_DIM // 128, 128)
    for q in range(_SEQ * _EMBED_DIM // 128):
        o_ref[q * 128:(q + 1) * 128, :] = _mxu_transpose(x3[:, q, :])


def _tc_out_body2(x_ref, prev_ref, o_ref):
    del prev_ref
    _tc_out_body(x_ref, o_ref)


def _tc_out_transpose(pieces, bb):
    sd = _SEQ * _EMBED_DIM
    np_ = len(pieces)
    hblk = _BATCH // np_ // bb
    t = pl.pallas_call(
        _tc_out_body,
        out_shape=jax.ShapeDtypeStruct((sd, _BATCH), jnp.float32),
        grid=(hblk,),
        in_specs=[pl.BlockSpec((bb * sd // 128, 128), lambda i: (i, 0))],
        out_specs=pl.BlockSpec((sd, bb), lambda i: (0, i)),
    )(pieces[0])
    for k in range(1, np_):
        t = pl.pallas_call(
            _tc_out_body2,
            out_shape=jax.ShapeDtypeStruct((sd, _BATCH), jnp.float32),
            grid=(hblk,),
            in_specs=[pl.BlockSpec((bb * sd // 128, 128), lambda i: (i, 0)),
                      pl.BlockSpec((8, 128), lambda i: (0, 0))],
            out_specs=pl.BlockSpec((sd, bb),
                                   lambda i, k=k: (0, i + k * hblk)),
            input_output_aliases={1: 0},
        )(pieces[k], t)
    return t


def _tc_table_body(x_ref, o_ref):
    y = jnp.transpose(x_ref[...], (1, 0))
    y3 = y.reshape(-1, 4, _EMBED_DIM)
    o_ref[...] = jnp.concatenate(
        [y3[:, a, :] for a in range(4)], axis=1)


def _tc_table_to_rowmajor(tt, bn):
    return pl.pallas_call(
        _tc_table_body,
        out_shape=jax.ShapeDtypeStruct((_VOCAB // 4, 128), jnp.float32),
        grid=((_VOCAB + bn - 1) // bn,),
        in_specs=[pl.BlockSpec((_EMBED_DIM, bn), lambda i: (0, i))],
        out_specs=pl.BlockSpec((bn // 4, 128), lambda i: (i, 0)),
    )(tt)


@jax.jit
def kernel(x, table):
    idx = x.reshape(-1)
    table = _tc_table_to_rowmajor(jnp.transpose(table, (1, 0)),
                                  8192).reshape(_VOCAB, _EMBED_DIM)
    mesh = plsc.VectorSubcoreMesh(core_axis_name="c", subcore_axis_name="s")

    def gather_half(idx_off):
        return pl.kernel(
            _make_sc_body(idx_off, _ROWS // 8 // _NW),
            out_type=jax.ShapeDtypeStruct((_ROWS // 8, _EMBED_DIM),
                                          jnp.float32),
            mesh=mesh,
            compiler_params=pltpu.CompilerParams(needs_layout_passes=False,
                                                 use_tc_tiling_on_sc=False),
            scratch_types=[
                pltpu.VMEM((_CHUNK,), jnp.int32),
                pltpu.VMEM((_CHUNK,), jnp.int32),
                pltpu.VMEM((_CHUNK, _EMBED_DIM), jnp.float32),
                pltpu.VMEM((_CHUNK, _EMBED_DIM), jnp.float32),
                pltpu.SemaphoreType.DMA,
                pltpu.SemaphoreType.DMA,
                pltpu.SemaphoreType.DMA,
                pltpu.SemaphoreType.DMA,
            ],
        )(idx, table)

    np_ = 8
    pieces = [gather_half(k * _ROWS // np_).reshape(_ROWS // np_ // 4, 128)
              for k in range(np_)]
    out2 = _tc_out_transpose(pieces, 128)
    return jnp.transpose(out2.reshape(_SEQ, _EMBED_DIM, _BATCH), (2, 0, 1))
